# Initial kernel scaffold; baseline (speedup 1.0000x reference)
import jax
import jax.numpy as jnp
from jax.experimental import pallas as pl


def kernel(seq_tokens, table, pe, gamma, beta):
    raise NotImplementedError("write your pallas kernel here")


# speedup vs baseline: 9.8156x; 9.8156x over previous
import functools

import jax
import jax.numpy as jnp
from jax import lax
from jax.experimental import pallas as pl
from jax.experimental.pallas import tpu as pltpu
from jax.experimental.pallas import tpu_sc as plsc

_B, _L, _V, _D = 1024, 1024, 21, 128
_EPS = 1e-5


_LB = 256


def _precompute_body(pe_ref, table_ref, gamma_ref, beta_ref, out_ref):
    pe_t = pe_ref[...]
    g = gamma_ref[...]
    b = beta_ref[...]
    for v in range(_V):
        x = pe_t + table_ref[pl.ds(v, 1), :]
        mean = jnp.mean(x, axis=1, keepdims=True)
        var = jnp.mean((x - mean) ** 2, axis=1, keepdims=True)
        y = (x - mean) / jnp.sqrt(var + _EPS)
        out_ref[v, :, :] = y * g + b


def _precompute(pe, table, gamma, beta):
    return pl.pallas_call(
        _precompute_body,
        grid=(_L // _LB,),
        in_specs=[
            pl.BlockSpec((_LB, _D), lambda i: (i, 0)),
            pl.BlockSpec((_V, _D), lambda i: (0, 0)),
            pl.BlockSpec((1, _D), lambda i: (0, 0)),
            pl.BlockSpec((1, _D), lambda i: (0, 0)),
        ],
        out_specs=pl.BlockSpec((_V, _LB, _D), lambda i: (0, i, 0)),
        out_shape=jax.ShapeDtypeStruct((_V, _L, _D), jnp.float32),
    )(pe, table, gamma.reshape(1, _D), beta.reshape(1, _D))


_NC, _NS = 2, 16
_NW = _NC * _NS
_SEQ_PER_W = _B // _NW
_CHUNK = 128
_NBUF = 4


@functools.cache
def _make_gather():
    mesh = plsc.VectorSubcoreMesh(core_axis_name="c", subcore_axis_name="s")

    @functools.partial(
        pl.kernel,
        mesh=mesh,
        out_type=jax.ShapeDtypeStruct((_B * _L // _CHUNK, _CHUNK, _D), jnp.float32),
        scratch_types=[
            pltpu.VMEM((_L,), jnp.int32),
            pltpu.VMEM((_L,), jnp.int32),
            pltpu.VMEM((_L,), jnp.int32),
            pltpu.VMEM((_NBUF, _CHUNK, _D), jnp.float32),
            pltpu.SemaphoreType.DMA,
        ],
    )
    def gather_k(tok_hbm, loff_hbm, precomp_hbm, out_hbm,
                 tok_v, loff_v, idx_v, rows_v, sem):
        wid = lax.axis_index("s") * _NC + lax.axis_index("c")
        pltpu.sync_copy(loff_hbm, loff_v)

        def seq_body(i, carry):
            seq = wid * _SEQ_PER_W + i
            base = seq * _L
            pltpu.sync_copy(tok_hbm.at[pl.ds(base, _L)], tok_v)
            for q in range(_L // 16):
                sl = pl.ds(q * 16, 16)
                idx_v[sl] = tok_v[sl] * _L + loff_v[sl]
            for h in range(_L // _CHUNK // _NBUF):
                copies = []
                for j in range(_NBUF):
                    isl = pl.ds((h * _NBUF + j) * _CHUNK, _CHUNK)
                    copies.append(
                        pltpu.async_copy(
                            precomp_hbm.at[idx_v.at[isl]], rows_v.at[j], sem))
                for c in copies:
                    c.wait()
                pltpu.sync_copy(
                    rows_v, out_hbm.at[pl.ds(seq * (_L // _CHUNK) + h * _NBUF, _NBUF)])
            return carry

        lax.fori_loop(0, _SEQ_PER_W, seq_body, 0)

    return gather_k


def kernel(seq_tokens, table, pe, gamma, beta):
    precomp = _precompute(pe, table, gamma, beta)
    precomp_flat = precomp.reshape(_V * _L, _D)
    tok_flat = seq_tokens.reshape(_B * _L)
    loff = jnp.arange(_L, dtype=jnp.int32)
    out = _make_gather()(tok_flat, loff, precomp_flat)
    return out.reshape(_B, _L, _D)


# speedup vs baseline: 10.7010x; 1.0902x over previous
import functools

import jax
import jax.numpy as jnp
from jax import lax
from jax.experimental import pallas as pl
from jax.experimental.pallas import tpu as pltpu
from jax.experimental.pallas import tpu_sc as plsc

_B, _L, _V, _D = 1024, 1024, 21, 128
_EPS = 1e-5


_LB = 256


def _precompute_body(pe_ref, table_ref, gamma_ref, beta_ref, out_ref):
    pe_t = pe_ref[...]
    g = gamma_ref[...]
    b = beta_ref[...]
    for v in range(_V):
        x = pe_t + table_ref[pl.ds(v, 1), :]
        mean = jnp.mean(x, axis=1, keepdims=True)
        var = jnp.mean((x - mean) ** 2, axis=1, keepdims=True)
        y = (x - mean) / jnp.sqrt(var + _EPS)
        out_ref[:, v, :] = y * g + b


def _precompute(pe, table, gamma, beta):
    return pl.pallas_call(
        _precompute_body,
        grid=(_L // _LB,),
        in_specs=[
            pl.BlockSpec((_LB, _D), lambda i: (i, 0)),
            pl.BlockSpec((_V, _D), lambda i: (0, 0)),
            pl.BlockSpec((1, _D), lambda i: (0, 0)),
            pl.BlockSpec((1, _D), lambda i: (0, 0)),
        ],
        out_specs=pl.BlockSpec((_LB, _V, _D), lambda i: (i, 0, 0)),
        out_shape=jax.ShapeDtypeStruct((_L, _V, _D), jnp.float32),
    )(pe, table, gamma.reshape(1, _D), beta.reshape(1, _D))


_NC, _NS = 2, 16
_LH = _L // _NC
_HROWS = _LH * _V
_BPW = _B // _NS
_CHUNK = 128
_NBUF = 2


@functools.cache
def _make_gather():
    mesh = plsc.VectorSubcoreMesh(core_axis_name="c", subcore_axis_name="s")

    @functools.partial(
        pl.kernel,
        mesh=mesh,
        out_type=jax.ShapeDtypeStruct((_B * _L // _CHUNK, _CHUNK, _D), jnp.float32),
        scratch_types=[
            pltpu.VMEM_SHARED((_HROWS, _D), jnp.float32),
            pltpu.VMEM((_LH,), jnp.int32),
            pltpu.VMEM((_LH,), jnp.int32),
            pltpu.VMEM((_LH,), jnp.int32),
            pltpu.VMEM((_NBUF, _CHUNK, _D), jnp.float32),
            pltpu.SemaphoreType.DMA,
        ],
    )
    def gather_k(tok_hbm, loff_hbm, precomp_hbm, out_hbm,
                 shared_v, tok_v, loff_v, idx_v, rows_v, sem):
        c = lax.axis_index("c")
        s = lax.axis_index("s")
        rows_per_sub = _HROWS // _NS
        pltpu.sync_copy(
            precomp_hbm.at[pl.ds(c * _HROWS + s * rows_per_sub, rows_per_sub)],
            shared_v.at[pl.ds(s * rows_per_sub, rows_per_sub)])
        pltpu.sync_copy(loff_hbm, loff_v)
        plsc.subcore_barrier()

        def seq_body(i, carry):
            b = s * _BPW + i
            base = b * _L + c * _LH
            pltpu.sync_copy(tok_hbm.at[pl.ds(base, _LH)], tok_v)
            for q in range(_LH // 16):
                sl = pl.ds(q * 16, 16)
                idx_v[sl] = tok_v[sl] + loff_v[sl]
            for h in range(_LH // _CHUNK // _NBUF):
                copies = []
                for j in range(_NBUF):
                    isl = pl.ds((h * _NBUF + j) * _CHUNK, _CHUNK)
                    copies.append(
                        pltpu.async_copy(
                            shared_v.at[idx_v.at[isl]], rows_v.at[j], sem))
                for cp in copies:
                    cp.wait()
                pltpu.sync_copy(
                    rows_v, out_hbm.at[pl.ds(base // _CHUNK + h * _NBUF, _NBUF)])
            return carry

        lax.fori_loop(0, _BPW, seq_body, 0)

    return gather_k


def kernel(seq_tokens, table, pe, gamma, beta):
    precomp = _precompute(pe, table, gamma, beta)
    precomp_flat = precomp.reshape(_L * _V, _D)
    tok_flat = seq_tokens.reshape(_B * _L)
    loff = jnp.arange(_LH, dtype=jnp.int32) * _V
    out = _make_gather()(tok_flat, loff, precomp_flat)
    return out.reshape(_B, _L, _D)


# speedup vs baseline: 17.1438x; 1.6021x over previous
import functools

import jax
import jax.numpy as jnp
from jax import lax
from jax.experimental import pallas as pl
from jax.experimental.pallas import tpu as pltpu
from jax.experimental.pallas import tpu_sc as plsc

_B, _L, _V, _D = 1024, 1024, 21, 128
_EPS = 1e-5


_LB = 256


def _precompute_body(pe_ref, table_ref, gamma_ref, beta_ref, out_ref):
    pe_t = pe_ref[...]
    g = gamma_ref[...]
    b = beta_ref[...]
    for v in range(_V):
        x = pe_t + table_ref[pl.ds(v, 1), :]
        mean = jnp.mean(x, axis=1, keepdims=True)
        var = jnp.mean((x - mean) ** 2, axis=1, keepdims=True)
        y = (x - mean) / jnp.sqrt(var + _EPS)
        out_ref[:, v, :] = y * g + b


def _precompute(pe, table, gamma, beta):
    return pl.pallas_call(
        _precompute_body,
        grid=(_L // _LB,),
        in_specs=[
            pl.BlockSpec((_LB, _D), lambda i: (i, 0)),
            pl.BlockSpec((_V, _D), lambda i: (0, 0)),
            pl.BlockSpec((1, _D), lambda i: (0, 0)),
            pl.BlockSpec((1, _D), lambda i: (0, 0)),
        ],
        out_specs=pl.BlockSpec((_LB, _V, _D), lambda i: (i, 0, 0)),
        out_shape=jax.ShapeDtypeStruct((_L, _V, _D), jnp.float32),
    )(pe, table, gamma.reshape(1, _D), beta.reshape(1, _D))


_NC, _NS = 2, 16
_LH = _L // _NC
_HROWS = _LH * _V
_BPW = _B // _NS
_CHUNK = 128
_GSEQ = 8
_CPS = _LH // _CHUNK
_GCH = _GSEQ * _CPS


@functools.cache
def _make_gather():
    mesh = plsc.VectorSubcoreMesh(core_axis_name="c", subcore_axis_name="s")

    @functools.partial(
        pl.kernel,
        mesh=mesh,
        out_type=jax.ShapeDtypeStruct((_B * _L // _CHUNK, _CHUNK, _D), jnp.float32),
        scratch_types=[
            pltpu.VMEM_SHARED((_HROWS, _D), jnp.float32),
            pltpu.VMEM((_GSEQ, _LH), jnp.int32),
            pltpu.VMEM((_LH,), jnp.int32),
            pltpu.VMEM((_GSEQ * _LH,), jnp.int32),
            pltpu.VMEM((_CHUNK, _D), jnp.float32),
            pltpu.VMEM((_CHUNK, _D), jnp.float32),
            pltpu.SemaphoreType.DMA,
            pltpu.SemaphoreType.DMA,
        ],
    )
    def gather_k(tok_hbm, loff_hbm, precomp_hbm, out_hbm,
                 shared_v, tok_v, loff_v, idx_v, rows_a, rows_b, semg, semw):
        c = lax.axis_index("c")
        s = lax.axis_index("s")
        rows_per_sub = _HROWS // _NS
        pltpu.sync_copy(
            precomp_hbm.at[pl.ds(c * _HROWS + s * rows_per_sub, rows_per_sub)],
            shared_v.at[pl.ds(s * rows_per_sub, rows_per_sub)])
        pltpu.sync_copy(loff_hbm, loff_v)
        plsc.subcore_barrier()

        rows = (rows_a, rows_b)

        def pending_write(buf):
            return pltpu.make_async_copy(buf, out_hbm.at[0], semw)

        def group_body(g, carry):
            b0 = s * _BPW + g * _GSEQ
            pltpu.sync_copy(
                tok_hbm.at[pl.ds(b0, _GSEQ), pl.ds(c * _LH, _LH)], tok_v)
            for r in range(_GSEQ):
                for q in range(_LH // 16):
                    sl = pl.ds(q * 16, 16)
                    idx_v[pl.ds(r * _LH + q * 16, 16)] = tok_v[r, sl] + loff_v[sl]

            @pl.when(g > 0)
            def _():
                pending_write(rows_a).wait()
                pending_write(rows_b).wait()

            gcp = [None, None]
            wcp = [None, None]
            for t in range(_GCH):
                p = t % 2
                if t >= 2:
                    wcp[p].wait()
                gcp[p] = pltpu.async_copy(
                    shared_v.at[idx_v.at[pl.ds(t * _CHUNK, _CHUNK)]],
                    rows[p], semg)
                if t >= 1:
                    q = (t - 1) % 2
                    r, j = divmod(t - 1, _CPS)
                    gcp[q].wait()
                    wcp[q] = pltpu.async_copy(
                        rows[q], out_hbm.at[(b0 + r) * (_L // _CHUNK) + c * _CPS + j],
                        semw)
            p = (_GCH - 1) % 2
            r, j = divmod(_GCH - 1, _CPS)
            gcp[p].wait()
            wcp[p] = pltpu.async_copy(
                rows[p], out_hbm.at[(b0 + r) * (_L // _CHUNK) + c * _CPS + j],
                semw)
            return carry

        lax.fori_loop(0, _BPW // _GSEQ, group_body, 0)
        pending_write(rows_a).wait()
        pending_write(rows_b).wait()

    return gather_k


def kernel(seq_tokens, table, pe, gamma, beta):
    precomp = _precompute(pe, table, gamma, beta)
    precomp_flat = precomp.reshape(_L * _V, _D)
    loff = jnp.arange(_LH, dtype=jnp.int32) * _V
    out = _make_gather()(seq_tokens, loff, precomp_flat)
    return out.reshape(_B, _L, _D)
